# Initial kernel scaffold; baseline (speedup 1.0000x reference)
import jax
import jax.numpy as jnp
from jax.experimental import pallas as pl


def kernel(x_user, x_movie, edge_index_um, edge_index_mu, Win_u, bin_u, Win_m, bin_m, Wl_um_0, bl_um_0, Wr_um_0, Wl_mu_0, bl_mu_0, Wr_mu_0, Wl_um_1, bl_um_1, Wr_um_1, Wl_mu_1, bl_mu_1, Wr_mu_1, Wout_u, bout_u, Wout_m, bout_m):
    raise NotImplementedError("write your pallas kernel here")


# speedup vs baseline: 3.8960x; 3.8960x over previous
import functools

import jax
import jax.numpy as jnp
from jax import lax
from jax.experimental import pallas as pl
from jax.experimental.pallas import tpu as pltpu
from jax.experimental.pallas import tpu_sc as plsc

N_NODES = 10000
N_EDGES = 160000
D_FEAT = 256
D_HID = 256
D_OUT = 128

NC = 2
NS = 16
EDGES_PER_SUB = N_EDGES // NS
CHUNK = 125
NCHUNKS = EDGES_PER_SUB // CHUNK
N_PAD = 10240
ROWS_PER_SUB = N_PAD // NS
ZBLK = 128
NZCOPY = ROWS_PER_SUB // ZBLK
NSTAGE = 2
CH_STG = NCHUNKS // NSTAGE


def _leaky(x):
    return jnp.where(x >= 0, x, 0.01 * x)


def _sc_agg(x2, src3, dst3):
    mesh = plsc.VectorSubcoreMesh(core_axis_name="c", subcore_axis_name="s")
    out_type = jax.ShapeDtypeStruct((NC, N_PAD, 128), jnp.float32)
    scratch = [
        pltpu.VMEM((CH_STG, CHUNK), jnp.int32),
        pltpu.VMEM((CH_STG, CHUNK), jnp.int32),
        pltpu.VMEM((ZBLK, 128), jnp.float32),
        pltpu.VMEM_SHARED((N_PAD, 128), jnp.float32),
    ]

    def body(x2_h, src_h, dst_h, o_h, src_v, dst_v, rows, acc):
        c = lax.axis_index("c")
        s = lax.axis_index("s")

        @pl.loop(0, ZBLK)
        def _zrow(i):
            @pl.loop(0, 128 // 16)
            def _zcol(k):
                rows[i, pl.ds(k * 16, 16)] = jnp.zeros((16,), jnp.float32)

        @pl.loop(0, NZCOPY)
        def _zacc(t):
            pltpu.sync_copy(
                rows, acc.at[pl.ds(s * ROWS_PER_SUB + t * ZBLK, ZBLK)])

        plsc.subcore_barrier()

        @pl.loop(0, NSTAGE)
        def _stage(h):
            pltpu.sync_copy(src_h.at[s].at[pl.ds(h * CH_STG, CH_STG)], src_v)
            pltpu.sync_copy(dst_h.at[s].at[pl.ds(h * CH_STG, CH_STG)], dst_v)

            @pl.loop(0, CH_STG)
            def _edge(j):
                pltpu.sync_copy(x2_h.at[c].at[src_v.at[j]],
                                rows.at[pl.ds(0, CHUNK)])
                pltpu.sync_copy(rows.at[pl.ds(0, CHUNK)],
                                acc.at[dst_v.at[j]], add=True)

        plsc.subcore_barrier()

        @pl.loop(0, NZCOPY)
        def _wr(t):
            r0 = s * ROWS_PER_SUB + t * ZBLK
            pltpu.sync_copy(acc.at[pl.ds(r0, ZBLK)],
                            o_h.at[c].at[pl.ds(r0, ZBLK)])

    fn = pl.kernel(body, out_type=out_type, mesh=mesh, scratch_types=scratch)
    return fn(x2, src3, dst3)[:, :N_NODES]


def _sc_deg(dst4):
    mesh = plsc.VectorSubcoreMesh(core_axis_name="c", subcore_axis_name="s")
    out_type = jax.ShapeDtypeStruct((NC, N_PAD, 128), jnp.float32)
    scratch = [
        pltpu.VMEM((NCHUNKS, CHUNK), jnp.int32),
        pltpu.VMEM((ZBLK, 128), jnp.float32),
        pltpu.VMEM_SHARED((N_PAD, 128), jnp.float32),
    ]

    def body(dst_h, o_h, dst_v, ones_v, acc):
        c = lax.axis_index("c")
        s = lax.axis_index("s")

        pltpu.sync_copy(dst_h.at[c].at[s], dst_v)

        @pl.loop(0, ZBLK)
        def _zrow(i):
            @pl.loop(0, 128 // 16)
            def _zcol(k):
                ones_v[i, pl.ds(k * 16, 16)] = jnp.zeros((16,), jnp.float32)

        @pl.loop(0, NZCOPY)
        def _zacc(t):
            pltpu.sync_copy(
                ones_v, acc.at[pl.ds(s * ROWS_PER_SUB + t * ZBLK, ZBLK)])

        @pl.loop(0, CHUNK)
        def _orow(i):
            @pl.loop(0, 128 // 16)
            def _ocol(k):
                ones_v[i, pl.ds(k * 16, 16)] = jnp.ones((16,), jnp.float32)

        plsc.subcore_barrier()

        @pl.loop(0, NCHUNKS)
        def _edge(j):
            pltpu.sync_copy(ones_v.at[pl.ds(0, CHUNK)],
                            acc.at[dst_v.at[j]], add=True)

        plsc.subcore_barrier()

        @pl.loop(0, NZCOPY)
        def _wr(t):
            r0 = s * ROWS_PER_SUB + t * ZBLK
            pltpu.sync_copy(acc.at[pl.ds(r0, ZBLK)],
                            o_h.at[c].at[pl.ds(r0, ZBLK)])

    fn = pl.kernel(body, out_type=out_type, mesh=mesh, scratch_types=scratch)
    return fn(dst4)[:, :N_NODES]


_BM = 1000


def _store_split(o_ref, h):
    o_ref[0] = h[:, :128]
    o_ref[1] = h[:, 128:]


def _tc_in_body(x_ref, w_ref, b_ref, o_ref):
    h = jnp.dot(x_ref[...], w_ref[...], preferred_element_type=jnp.float32)
    h = _leaky(h + b_ref[...])
    _store_split(o_ref, h)


def _tc_in(x, w, b):
    grid = (N_NODES // _BM,)
    return pl.pallas_call(
        _tc_in_body,
        grid=grid,
        in_specs=[
            pl.BlockSpec((_BM, D_FEAT), lambda i: (i, 0)),
            pl.BlockSpec((D_FEAT, D_HID), lambda i: (0, 0)),
            pl.BlockSpec((1, D_HID), lambda i: (0, 0)),
        ],
        out_specs=pl.BlockSpec((2, _BM, 128), lambda i: (0, i, 0)),
        out_shape=jax.ShapeDtypeStruct((2, N_NODES, 128), jnp.float32),
    )(x, w, b.reshape(1, -1))


def _tc_sage_body(summ_ref, deg_ref, xd_ref, w_ref, b_ref, o_ref):
    scale = 1.0 / jnp.maximum(deg_ref[:, 0:1], 1.0)
    a = jnp.concatenate(
        [summ_ref[0] * scale, summ_ref[1] * scale, xd_ref[0], xd_ref[1]],
        axis=1)
    h = jnp.dot(a, w_ref[...], preferred_element_type=jnp.float32)
    h = _leaky(h + b_ref[...])
    _store_split(o_ref, h)


def _tc_sage(summ2, deg, xd2, wl, bl, wr):
    wcat = jnp.concatenate([wl, wr], axis=0)
    grid = (N_NODES // _BM,)
    return pl.pallas_call(
        _tc_sage_body,
        grid=grid,
        in_specs=[
            pl.BlockSpec((2, _BM, 128), lambda i: (0, i, 0)),
            pl.BlockSpec((_BM, 128), lambda i: (i, 0)),
            pl.BlockSpec((2, _BM, 128), lambda i: (0, i, 0)),
            pl.BlockSpec((2 * D_HID, D_HID), lambda i: (0, 0)),
            pl.BlockSpec((1, D_HID), lambda i: (0, 0)),
        ],
        out_specs=pl.BlockSpec((2, _BM, 128), lambda i: (0, i, 0)),
        out_shape=jax.ShapeDtypeStruct((2, N_NODES, 128), jnp.float32),
    )(summ2, deg, xd2, wcat, bl.reshape(1, -1))


def _tc_out_body(x_ref, w_ref, b_ref, o_ref):
    a = jnp.concatenate([x_ref[0], x_ref[1]], axis=1)
    o_ref[...] = jnp.dot(a, w_ref[...],
                         preferred_element_type=jnp.float32) + b_ref[...]


def _tc_out(x2, w, b):
    grid = (N_NODES // _BM,)
    return pl.pallas_call(
        _tc_out_body,
        grid=grid,
        in_specs=[
            pl.BlockSpec((2, _BM, 128), lambda i: (0, i, 0)),
            pl.BlockSpec((D_HID, D_OUT), lambda i: (0, 0)),
            pl.BlockSpec((1, D_OUT), lambda i: (0, 0)),
        ],
        out_specs=pl.BlockSpec((_BM, D_OUT), lambda i: (i, 0)),
        out_shape=jax.ShapeDtypeStruct((N_NODES, D_OUT), jnp.float32),
    )(x2, w, b.reshape(1, -1))


def kernel(x_user, x_movie, edge_index_um, edge_index_mu,
           Win_u, bin_u, Win_m, bin_m,
           Wl_um_0, bl_um_0, Wr_um_0, Wl_mu_0, bl_mu_0, Wr_mu_0,
           Wl_um_1, bl_um_1, Wr_um_1, Wl_mu_1, bl_mu_1, Wr_mu_1,
           Wout_u, bout_u, Wout_m, bout_m):
    src_um = edge_index_um[0].reshape(NS, NCHUNKS, CHUNK)
    dst_um = edge_index_um[1].reshape(NS, NCHUNKS, CHUNK)
    src_mu = edge_index_mu[0].reshape(NS, NCHUNKS, CHUNK)
    dst_mu = edge_index_mu[1].reshape(NS, NCHUNKS, CHUNK)

    xu2 = _tc_in(x_user, Win_u, bin_u)
    xm2 = _tc_in(x_movie, Win_m, bin_m)

    deg2 = _sc_deg(jnp.stack([dst_um, dst_mu], axis=0))
    deg_m = deg2[0]
    deg_u = deg2[1]

    summ_m = _sc_agg(xu2, src_um, dst_um)
    summ_u = _sc_agg(xm2, src_mu, dst_mu)
    xm2 = _tc_sage(summ_m, deg_m, xm2, Wl_um_0, bl_um_0, Wr_um_0)
    xu2 = _tc_sage(summ_u, deg_u, xu2, Wl_mu_0, bl_mu_0, Wr_mu_0)

    summ_m = _sc_agg(xu2, src_um, dst_um)
    summ_u = _sc_agg(xm2, src_mu, dst_mu)
    xm2 = _tc_sage(summ_m, deg_m, xm2, Wl_um_1, bl_um_1, Wr_um_1)
    xu2 = _tc_sage(summ_u, deg_u, xu2, Wl_mu_1, bl_mu_1, Wr_mu_1)

    out_u = _tc_out(xu2, Wout_u, bout_u)
    out_m = _tc_out(xm2, Wout_m, bout_m)
    return (out_u, out_m)


# speedup vs baseline: 4.8224x; 1.2378x over previous
import functools

import jax
import jax.numpy as jnp
from jax import lax
from jax.experimental import pallas as pl
from jax.experimental.pallas import tpu as pltpu
from jax.experimental.pallas import tpu_sc as plsc

N_NODES = 10000
N_EDGES = 160000
D_FEAT = 256
D_HID = 256
D_OUT = 128

NC = 2
NS = 16
EDGES_PER_SUB = N_EDGES // NS
CHUNK = 100
NCHUNKS = EDGES_PER_SUB // CHUNK
N_PAD = 10240
ROWS_PER_SUB = N_PAD // NS
ZBLK = 128
NZCOPY = ROWS_PER_SUB // ZBLK
NSTAGE = 5
CH_STG = NCHUNKS // NSTAGE


def _leaky(x):
    return jnp.where(x >= 0, x, 0.01 * x)


def _sc_agg(x2, src3, dst3, zeros_h):
    mesh = plsc.VectorSubcoreMesh(core_axis_name="c", subcore_axis_name="s")
    out_type = jax.ShapeDtypeStruct((NC, N_PAD, 128), jnp.float32)
    scratch = [
        pltpu.VMEM((CH_STG, CHUNK), jnp.int32),
        pltpu.VMEM((CH_STG, CHUNK), jnp.int32),
        pltpu.VMEM((CHUNK, 128), jnp.float32),
        pltpu.VMEM((CHUNK, 128), jnp.float32),
        pltpu.SemaphoreType.DMA,
        pltpu.SemaphoreType.DMA,
        pltpu.VMEM_SHARED((N_PAD, 128), jnp.float32),
    ]

    def body(x2_h, src_h, dst_h, z_h, o_h,
             src_v, dst_v, rows0, rows1, sem0, sem1, acc):
        c = lax.axis_index("c")
        s = lax.axis_index("s")
        rows = (rows0, rows1)
        sems = (sem0, sem1)

        @pl.loop(0, NZCOPY)
        def _zacc(t):
            pltpu.sync_copy(
                z_h, acc.at[pl.ds(s * ROWS_PER_SUB + t * ZBLK, ZBLK)])

        plsc.subcore_barrier()

        def _gather(j, b):
            pltpu.async_copy(x2_h.at[c].at[src_v.at[j]], rows[b], sems[b])

        def _scat(j, b):
            pltpu.make_async_copy(x2_h.at[c].at[src_v.at[j]],
                                  rows[b], sems[b]).wait()
            pltpu.sync_copy(rows[b], acc.at[dst_v.at[j]], add=True)

        @pl.loop(0, NSTAGE)
        def _stage(h):
            pltpu.sync_copy(src_h.at[s].at[h], src_v)
            pltpu.sync_copy(dst_h.at[s].at[h], dst_v)

            _gather(0, 0)

            @pl.loop(0, CH_STG // 2 - 1)
            def _pipe(g):
                _gather(2 * g + 1, 1)
                _scat(2 * g, 0)
                _gather(2 * g + 2, 0)
                _scat(2 * g + 1, 1)

            _gather(CH_STG - 1, 1)
            _scat(CH_STG - 2, 0)
            _scat(CH_STG - 1, 1)

        plsc.subcore_barrier()

        @pl.loop(0, NZCOPY)
        def _wr(t):
            r0 = s * ROWS_PER_SUB + t * ZBLK
            pltpu.sync_copy(acc.at[pl.ds(r0, ZBLK)],
                            o_h.at[c].at[pl.ds(r0, ZBLK)])

    fn = pl.kernel(body, out_type=out_type, mesh=mesh, scratch_types=scratch)
    return fn(x2, src3, dst3, zeros_h)[:, :N_NODES]


def _sc_deg(dst4):
    mesh = plsc.VectorSubcoreMesh(core_axis_name="c", subcore_axis_name="s")
    out_type = jax.ShapeDtypeStruct((NC, N_PAD, 128), jnp.float32)
    scratch = [
        pltpu.VMEM((NSTAGE, CH_STG, CHUNK), jnp.int32),
        pltpu.VMEM((ZBLK, 128), jnp.float32),
        pltpu.VMEM_SHARED((N_PAD, 128), jnp.float32),
    ]

    def body(dst_h, o_h, dst_v, ones_v, acc):
        c = lax.axis_index("c")
        s = lax.axis_index("s")

        pltpu.sync_copy(dst_h.at[c].at[s], dst_v)

        @pl.loop(0, ZBLK)
        def _zrow(i):
            @pl.loop(0, 128 // 16)
            def _zcol(k):
                ones_v[i, pl.ds(k * 16, 16)] = jnp.zeros((16,), jnp.float32)

        @pl.loop(0, NZCOPY)
        def _zacc(t):
            pltpu.sync_copy(
                ones_v, acc.at[pl.ds(s * ROWS_PER_SUB + t * ZBLK, ZBLK)])

        @pl.loop(0, CHUNK)
        def _orow(i):
            @pl.loop(0, 128 // 16)
            def _ocol(k):
                ones_v[i, pl.ds(k * 16, 16)] = jnp.ones((16,), jnp.float32)

        plsc.subcore_barrier()

        @pl.loop(0, NSTAGE)
        def _stage(h):
            @pl.loop(0, CH_STG)
            def _edge(j):
                pltpu.sync_copy(ones_v.at[pl.ds(0, CHUNK)],
                                acc.at[dst_v.at[h].at[j]], add=True)

        plsc.subcore_barrier()

        @pl.loop(0, NZCOPY)
        def _wr(t):
            r0 = s * ROWS_PER_SUB + t * ZBLK
            pltpu.sync_copy(acc.at[pl.ds(r0, ZBLK)],
                            o_h.at[c].at[pl.ds(r0, ZBLK)])

    fn = pl.kernel(body, out_type=out_type, mesh=mesh, scratch_types=scratch)
    return fn(dst4)[:, :N_NODES]


_BM = 1000


def _store_split(o_ref, h):
    o_ref[0] = h[:, :128]
    o_ref[1] = h[:, 128:]


def _tc_in_body(x_ref, w_ref, b_ref, o_ref):
    h = jnp.dot(x_ref[...], w_ref[...], preferred_element_type=jnp.float32)
    h = _leaky(h + b_ref[...])
    _store_split(o_ref, h)


def _tc_in(x, w, b):
    grid = (N_NODES // _BM,)
    return pl.pallas_call(
        _tc_in_body,
        grid=grid,
        in_specs=[
            pl.BlockSpec((_BM, D_FEAT), lambda i: (i, 0)),
            pl.BlockSpec((D_FEAT, D_HID), lambda i: (0, 0)),
            pl.BlockSpec((1, D_HID), lambda i: (0, 0)),
        ],
        out_specs=pl.BlockSpec((2, _BM, 128), lambda i: (0, i, 0)),
        out_shape=jax.ShapeDtypeStruct((2, N_NODES, 128), jnp.float32),
    )(x, w, b.reshape(1, -1))


def _tc_sage_body(summ_ref, deg_ref, xd_ref, w_ref, b_ref, o_ref):
    scale = 1.0 / jnp.maximum(deg_ref[:, 0:1], 1.0)
    a = jnp.concatenate(
        [summ_ref[0] * scale, summ_ref[1] * scale, xd_ref[0], xd_ref[1]],
        axis=1)
    h = jnp.dot(a, w_ref[...], preferred_element_type=jnp.float32)
    h = _leaky(h + b_ref[...])
    _store_split(o_ref, h)


def _tc_sage(summ2, deg, xd2, wl, bl, wr):
    wcat = jnp.concatenate([wl, wr], axis=0)
    grid = (N_NODES // _BM,)
    return pl.pallas_call(
        _tc_sage_body,
        grid=grid,
        in_specs=[
            pl.BlockSpec((2, _BM, 128), lambda i: (0, i, 0)),
            pl.BlockSpec((_BM, 128), lambda i: (i, 0)),
            pl.BlockSpec((2, _BM, 128), lambda i: (0, i, 0)),
            pl.BlockSpec((2 * D_HID, D_HID), lambda i: (0, 0)),
            pl.BlockSpec((1, D_HID), lambda i: (0, 0)),
        ],
        out_specs=pl.BlockSpec((2, _BM, 128), lambda i: (0, i, 0)),
        out_shape=jax.ShapeDtypeStruct((2, N_NODES, 128), jnp.float32),
    )(summ2, deg, xd2, wcat, bl.reshape(1, -1))


def _tc_out_body(x_ref, w_ref, b_ref, o_ref):
    a = jnp.concatenate([x_ref[0], x_ref[1]], axis=1)
    o_ref[...] = jnp.dot(a, w_ref[...],
                         preferred_element_type=jnp.float32) + b_ref[...]


def _tc_out(x2, w, b):
    grid = (N_NODES // _BM,)
    return pl.pallas_call(
        _tc_out_body,
        grid=grid,
        in_specs=[
            pl.BlockSpec((2, _BM, 128), lambda i: (0, i, 0)),
            pl.BlockSpec((D_HID, D_OUT), lambda i: (0, 0)),
            pl.BlockSpec((1, D_OUT), lambda i: (0, 0)),
        ],
        out_specs=pl.BlockSpec((_BM, D_OUT), lambda i: (i, 0)),
        out_shape=jax.ShapeDtypeStruct((N_NODES, D_OUT), jnp.float32),
    )(x2, w, b.reshape(1, -1))


def kernel(x_user, x_movie, edge_index_um, edge_index_mu,
           Win_u, bin_u, Win_m, bin_m,
           Wl_um_0, bl_um_0, Wr_um_0, Wl_mu_0, bl_mu_0, Wr_mu_0,
           Wl_um_1, bl_um_1, Wr_um_1, Wl_mu_1, bl_mu_1, Wr_mu_1,
           Wout_u, bout_u, Wout_m, bout_m):
    src_um = edge_index_um[0].reshape(NS, NSTAGE, CH_STG, CHUNK)
    dst_um = edge_index_um[1].reshape(NS, NSTAGE, CH_STG, CHUNK)
    src_mu = edge_index_mu[0].reshape(NS, NSTAGE, CH_STG, CHUNK)
    dst_mu = edge_index_mu[1].reshape(NS, NSTAGE, CH_STG, CHUNK)

    zeros_blk = jnp.zeros((ZBLK, 128), jnp.float32)

    xu2 = _tc_in(x_user, Win_u, bin_u)
    xm2 = _tc_in(x_movie, Win_m, bin_m)

    deg2 = _sc_deg(jnp.stack([dst_um, dst_mu], axis=0))
    deg_m = deg2[0]
    deg_u = deg2[1]

    summ_m = _sc_agg(xu2, src_um, dst_um, zeros_blk)
    summ_u = _sc_agg(xm2, src_mu, dst_mu, zeros_blk)
    xm2 = _tc_sage(summ_m, deg_m, xm2, Wl_um_0, bl_um_0, Wr_um_0)
    xu2 = _tc_sage(summ_u, deg_u, xu2, Wl_mu_0, bl_mu_0, Wr_mu_0)

    summ_m = _sc_agg(xu2, src_um, dst_um, zeros_blk)
    summ_u = _sc_agg(xm2, src_mu, dst_mu, zeros_blk)
    xm2 = _tc_sage(summ_m, deg_m, xm2, Wl_um_1, bl_um_1, Wr_um_1)
    xu2 = _tc_sage(summ_u, deg_u, xu2, Wl_mu_1, bl_mu_1, Wr_mu_1)

    out_u = _tc_out(xu2, Wout_u, bout_u)
    out_m = _tc_out(xm2, Wout_m, bout_m)
    return (out_u, out_m)


# speedup vs baseline: 5.1769x; 1.0735x over previous
import functools

import jax
import jax.numpy as jnp
from jax import lax
from jax.experimental import pallas as pl
from jax.experimental.pallas import tpu as pltpu
from jax.experimental.pallas import tpu_sc as plsc

N_NODES = 10000
N_EDGES = 160000
D_FEAT = 256
D_HID = 256
D_OUT = 128

NC = 2
NS = 16
EDGES_PER_SUB = N_EDGES // NS
CHUNK = 50
NCHUNKS = EDGES_PER_SUB // CHUNK
N_PAD = 10240
ROWS_PER_SUB = N_PAD // NS
ZBLK = 128
NZCOPY = ROWS_PER_SUB // ZBLK
NSTAGE = 5
CH_STG = NCHUNKS // NSTAGE
NBUF = 4


def _leaky(x):
    return jnp.where(x >= 0, x, 0.01 * x)


def _sc_agg(x2, src3, dst3, zeros_h):
    mesh = plsc.VectorSubcoreMesh(core_axis_name="c", subcore_axis_name="s")
    out_type = jax.ShapeDtypeStruct((NC, N_PAD, 128), jnp.float32)
    scratch = (
        [pltpu.VMEM((CH_STG, CHUNK), jnp.int32),
         pltpu.VMEM((CH_STG, CHUNK), jnp.int32)]
        + [pltpu.VMEM((CHUNK, 128), jnp.float32) for _ in range(NBUF)]
        + [pltpu.SemaphoreType.DMA for _ in range(NBUF)]
        + [pltpu.VMEM_SHARED((N_PAD, 128), jnp.float32)]
    )

    def body(x2_h, src_h, dst_h, z_h, o_h, src_v, dst_v, *rest):
        rows = rest[:NBUF]
        sems = rest[NBUF:2 * NBUF]
        acc = rest[2 * NBUF]
        c = lax.axis_index("c")
        s = lax.axis_index("s")

        @pl.loop(0, NZCOPY)
        def _zacc(t):
            pltpu.sync_copy(
                z_h, acc.at[pl.ds(s * ROWS_PER_SUB + t * ZBLK, ZBLK)])

        plsc.subcore_barrier()

        def _gather(j, b):
            pltpu.async_copy(x2_h.at[c].at[src_v.at[j]], rows[b], sems[b])

        def _scat(j, b):
            pltpu.make_async_copy(x2_h.at[c].at[src_v.at[j]],
                                  rows[b], sems[b]).wait()
            pltpu.sync_copy(rows[b], acc.at[dst_v.at[j]], add=True)

        @pl.loop(0, NSTAGE)
        def _stage(h):
            pltpu.sync_copy(src_h.at[s].at[h], src_v)
            pltpu.sync_copy(dst_h.at[s].at[h], dst_v)

            for b in range(NBUF - 1):
                _gather(b, b)

            @pl.loop(0, CH_STG // NBUF)
            def _pipe(q):
                for b in range(NBUF):
                    j = NBUF * q + b
                    nxt = j + NBUF - 1

                    @pl.when(nxt < CH_STG)
                    def _():
                        _gather(nxt, (b + NBUF - 1) % NBUF)

                    _scat(j, b)

        plsc.subcore_barrier()

        @pl.loop(0, NZCOPY)
        def _wr(t):
            r0 = s * ROWS_PER_SUB + t * ZBLK
            pltpu.sync_copy(acc.at[pl.ds(r0, ZBLK)],
                            o_h.at[c].at[pl.ds(r0, ZBLK)])

    fn = pl.kernel(body, out_type=out_type, mesh=mesh, scratch_types=scratch)
    return fn(x2, src3, dst3, zeros_h)[:, :N_NODES]


def _sc_deg(dst4):
    mesh = plsc.VectorSubcoreMesh(core_axis_name="c", subcore_axis_name="s")
    out_type = jax.ShapeDtypeStruct((NC, N_PAD, 128), jnp.float32)
    scratch = [
        pltpu.VMEM((NSTAGE, CH_STG, CHUNK), jnp.int32),
        pltpu.VMEM((ZBLK, 128), jnp.float32),
        pltpu.VMEM_SHARED((N_PAD, 128), jnp.float32),
    ]

    def body(dst_h, o_h, dst_v, ones_v, acc):
        c = lax.axis_index("c")
        s = lax.axis_index("s")

        pltpu.sync_copy(dst_h.at[c].at[s], dst_v)

        @pl.loop(0, ZBLK)
        def _zrow(i):
            @pl.loop(0, 128 // 16)
            def _zcol(k):
                ones_v[i, pl.ds(k * 16, 16)] = jnp.zeros((16,), jnp.float32)

        @pl.loop(0, NZCOPY)
        def _zacc(t):
            pltpu.sync_copy(
                ones_v, acc.at[pl.ds(s * ROWS_PER_SUB + t * ZBLK, ZBLK)])

        @pl.loop(0, CHUNK)
        def _orow(i):
            @pl.loop(0, 128 // 16)
            def _ocol(k):
                ones_v[i, pl.ds(k * 16, 16)] = jnp.ones((16,), jnp.float32)

        plsc.subcore_barrier()

        @pl.loop(0, NSTAGE)
        def _stage(h):
            @pl.loop(0, CH_STG)
            def _edge(j):
                pltpu.sync_copy(ones_v.at[pl.ds(0, CHUNK)],
                                acc.at[dst_v.at[h].at[j]], add=True)

        plsc.subcore_barrier()

        @pl.loop(0, NZCOPY)
        def _wr(t):
            r0 = s * ROWS_PER_SUB + t * ZBLK
            pltpu.sync_copy(acc.at[pl.ds(r0, ZBLK)],
                            o_h.at[c].at[pl.ds(r0, ZBLK)])

    fn = pl.kernel(body, out_type=out_type, mesh=mesh, scratch_types=scratch)
    return fn(dst4)[:, :N_NODES]


_BM = 1000


def _store_split(o_ref, h):
    o_ref[0] = h[:, :128]
    o_ref[1] = h[:, 128:]


def _tc_in_body(x_ref, w_ref, b_ref, o_ref):
    h = jnp.dot(x_ref[...], w_ref[...], preferred_element_type=jnp.float32)
    h = _leaky(h + b_ref[...])
    _store_split(o_ref, h)


def _tc_in(x, w, b):
    grid = (N_NODES // _BM,)
    return pl.pallas_call(
        _tc_in_body,
        grid=grid,
        in_specs=[
            pl.BlockSpec((_BM, D_FEAT), lambda i: (i, 0)),
            pl.BlockSpec((D_FEAT, D_HID), lambda i: (0, 0)),
            pl.BlockSpec((1, D_HID), lambda i: (0, 0)),
        ],
        out_specs=pl.BlockSpec((2, _BM, 128), lambda i: (0, i, 0)),
        out_shape=jax.ShapeDtypeStruct((2, N_NODES, 128), jnp.float32),
    )(x, w, b.reshape(1, -1))


def _tc_sage_body(summ_ref, deg_ref, xd_ref, w_ref, b_ref, o_ref):
    scale = 1.0 / jnp.maximum(deg_ref[:, 0:1], 1.0)
    a = jnp.concatenate(
        [summ_ref[0] * scale, summ_ref[1] * scale, xd_ref[0], xd_ref[1]],
        axis=1)
    h = jnp.dot(a, w_ref[...], preferred_element_type=jnp.float32)
    h = _leaky(h + b_ref[...])
    _store_split(o_ref, h)


def _tc_sage(summ2, deg, xd2, wl, bl, wr):
    wcat = jnp.concatenate([wl, wr], axis=0)
    grid = (N_NODES // _BM,)
    return pl.pallas_call(
        _tc_sage_body,
        grid=grid,
        in_specs=[
            pl.BlockSpec((2, _BM, 128), lambda i: (0, i, 0)),
            pl.BlockSpec((_BM, 128), lambda i: (i, 0)),
            pl.BlockSpec((2, _BM, 128), lambda i: (0, i, 0)),
            pl.BlockSpec((2 * D_HID, D_HID), lambda i: (0, 0)),
            pl.BlockSpec((1, D_HID), lambda i: (0, 0)),
        ],
        out_specs=pl.BlockSpec((2, _BM, 128), lambda i: (0, i, 0)),
        out_shape=jax.ShapeDtypeStruct((2, N_NODES, 128), jnp.float32),
    )(summ2, deg, xd2, wcat, bl.reshape(1, -1))


def _tc_out_body(x_ref, w_ref, b_ref, o_ref):
    a = jnp.concatenate([x_ref[0], x_ref[1]], axis=1)
    o_ref[...] = jnp.dot(a, w_ref[...],
                         preferred_element_type=jnp.float32) + b_ref[...]


def _tc_out(x2, w, b):
    grid = (N_NODES // _BM,)
    return pl.pallas_call(
        _tc_out_body,
        grid=grid,
        in_specs=[
            pl.BlockSpec((2, _BM, 128), lambda i: (0, i, 0)),
            pl.BlockSpec((D_HID, D_OUT), lambda i: (0, 0)),
            pl.BlockSpec((1, D_OUT), lambda i: (0, 0)),
        ],
        out_specs=pl.BlockSpec((_BM, D_OUT), lambda i: (i, 0)),
        out_shape=jax.ShapeDtypeStruct((N_NODES, D_OUT), jnp.float32),
    )(x2, w, b.reshape(1, -1))


def kernel(x_user, x_movie, edge_index_um, edge_index_mu,
           Win_u, bin_u, Win_m, bin_m,
           Wl_um_0, bl_um_0, Wr_um_0, Wl_mu_0, bl_mu_0, Wr_mu_0,
           Wl_um_1, bl_um_1, Wr_um_1, Wl_mu_1, bl_mu_1, Wr_mu_1,
           Wout_u, bout_u, Wout_m, bout_m):
    src_um = edge_index_um[0].reshape(NS, NSTAGE, CH_STG, CHUNK)
    dst_um = edge_index_um[1].reshape(NS, NSTAGE, CH_STG, CHUNK)
    src_mu = edge_index_mu[0].reshape(NS, NSTAGE, CH_STG, CHUNK)
    dst_mu = edge_index_mu[1].reshape(NS, NSTAGE, CH_STG, CHUNK)

    zeros_blk = jnp.zeros((ZBLK, 128), jnp.float32)

    xu2 = _tc_in(x_user, Win_u, bin_u)
    xm2 = _tc_in(x_movie, Win_m, bin_m)

    deg2 = _sc_deg(jnp.stack([dst_um, dst_mu], axis=0))
    deg_m = deg2[0]
    deg_u = deg2[1]

    summ_m = _sc_agg(xu2, src_um, dst_um, zeros_blk)
    summ_u = _sc_agg(xm2, src_mu, dst_mu, zeros_blk)
    xm2 = _tc_sage(summ_m, deg_m, xm2, Wl_um_0, bl_um_0, Wr_um_0)
    xu2 = _tc_sage(summ_u, deg_u, xu2, Wl_mu_0, bl_mu_0, Wr_mu_0)

    summ_m = _sc_agg(xu2, src_um, dst_um, zeros_blk)
    summ_u = _sc_agg(xm2, src_mu, dst_mu, zeros_blk)
    xm2 = _tc_sage(summ_m, deg_m, xm2, Wl_um_1, bl_um_1, Wr_um_1)
    xu2 = _tc_sage(summ_u, deg_u, xu2, Wl_mu_1, bl_mu_1, Wr_mu_1)

    out_u = _tc_out(xu2, Wout_u, bout_u)
    out_m = _tc_out(xm2, Wout_m, bout_m)
    return (out_u, out_m)


# speedup vs baseline: 5.2973x; 1.0233x over previous
import functools

import jax
import jax.numpy as jnp
from jax import lax
from jax.experimental import pallas as pl
from jax.experimental.pallas import tpu as pltpu
from jax.experimental.pallas import tpu_sc as plsc

N_NODES = 10000
N_EDGES = 160000
D_FEAT = 256
D_HID = 256
D_OUT = 128

NC = 2
NS = 16
EDGES_PER_SUB = N_EDGES // NS
CHUNK = 50
NCHUNKS = EDGES_PER_SUB // CHUNK
N_PAD = 10240
ROWS_PER_SUB = N_PAD // NS
ZBLK = 128
NZCOPY = ROWS_PER_SUB // ZBLK
NSTAGE = 5
CH_STG = NCHUNKS // NSTAGE
NBUF = 4


def _leaky(x):
    return jnp.where(x >= 0, x, 0.01 * x)


def _sc_agg(x2, src3, dst3, zeros_h):
    mesh = plsc.VectorSubcoreMesh(core_axis_name="c", subcore_axis_name="s")
    out_type = jax.ShapeDtypeStruct((NC, N_PAD, 128), jnp.float32)
    scratch = (
        [pltpu.VMEM((CH_STG, CHUNK), jnp.int32),
         pltpu.VMEM((CH_STG, CHUNK), jnp.int32)]
        + [pltpu.VMEM((CHUNK, 128), jnp.float32) for _ in range(NBUF)]
        + [pltpu.SemaphoreType.DMA for _ in range(NBUF)]
        + [pltpu.VMEM_SHARED((N_PAD, 128), jnp.float32)]
    )

    def body(x2_h, src_h, dst_h, z_h, o_h, src_v, dst_v, *rest):
        rows = rest[:NBUF]
        sems = rest[NBUF:2 * NBUF]
        acc = rest[2 * NBUF]
        c = lax.axis_index("c")
        s = lax.axis_index("s")

        @pl.loop(0, NZCOPY)
        def _zacc(t):
            pltpu.sync_copy(
                z_h, acc.at[pl.ds(s * ROWS_PER_SUB + t * ZBLK, ZBLK)])

        plsc.subcore_barrier()

        def _gather(j, b):
            pltpu.async_copy(x2_h.at[c].at[src_v.at[j]], rows[b], sems[b])

        def _scat(j, b):
            pltpu.make_async_copy(x2_h.at[c].at[src_v.at[j]],
                                  rows[b], sems[b]).wait()
            pltpu.sync_copy(rows[b], acc.at[dst_v.at[j]], add=True)

        @pl.loop(0, NSTAGE)
        def _stage(h):
            pltpu.sync_copy(src_h.at[s].at[h], src_v)
            pltpu.sync_copy(dst_h.at[s].at[h], dst_v)

            for b in range(NBUF - 1):
                _gather(b, b)

            @pl.loop(0, CH_STG // NBUF)
            def _pipe(q):
                for b in range(NBUF):
                    j = NBUF * q + b
                    nxt = j + NBUF - 1

                    @pl.when(nxt < CH_STG)
                    def _():
                        _gather(nxt, (b + NBUF - 1) % NBUF)

                    _scat(j, b)

        plsc.subcore_barrier()

        @pl.loop(0, NZCOPY)
        def _wr(t):
            r0 = s * ROWS_PER_SUB + t * ZBLK
            pltpu.sync_copy(acc.at[pl.ds(r0, ZBLK)],
                            o_h.at[c].at[pl.ds(r0, ZBLK)])

    fn = pl.kernel(body, out_type=out_type, mesh=mesh, scratch_types=scratch)
    return fn(x2, src3, dst3, zeros_h)[:, :N_NODES]


def _sc_deg(dst4):
    mesh = plsc.VectorSubcoreMesh(core_axis_name="c", subcore_axis_name="s")
    out_type = jax.ShapeDtypeStruct((NC, N_PAD, 128), jnp.float32)
    scratch = [
        pltpu.VMEM((NSTAGE, CH_STG, CHUNK), jnp.int32),
        pltpu.VMEM((ZBLK, 128), jnp.float32),
        pltpu.VMEM_SHARED((N_PAD, 128), jnp.float32),
    ]

    def body(dst_h, o_h, dst_v, ones_v, acc):
        c = lax.axis_index("c")
        s = lax.axis_index("s")

        pltpu.sync_copy(dst_h.at[c].at[s], dst_v)

        @pl.loop(0, ZBLK)
        def _zrow(i):
            @pl.loop(0, 128 // 16)
            def _zcol(k):
                ones_v[i, pl.ds(k * 16, 16)] = jnp.zeros((16,), jnp.float32)

        @pl.loop(0, NZCOPY)
        def _zacc(t):
            pltpu.sync_copy(
                ones_v, acc.at[pl.ds(s * ROWS_PER_SUB + t * ZBLK, ZBLK)])

        @pl.loop(0, CHUNK)
        def _orow(i):
            @pl.loop(0, 128 // 16)
            def _ocol(k):
                ones_v[i, pl.ds(k * 16, 16)] = jnp.ones((16,), jnp.float32)

        plsc.subcore_barrier()

        @pl.loop(0, NSTAGE)
        def _stage(h):
            @pl.loop(0, CH_STG)
            def _edge(j):
                pltpu.sync_copy(ones_v.at[pl.ds(0, CHUNK)],
                                acc.at[dst_v.at[h].at[j]], add=True)

        plsc.subcore_barrier()

        @pl.loop(0, NZCOPY)
        def _wr(t):
            r0 = s * ROWS_PER_SUB + t * ZBLK
            pltpu.sync_copy(acc.at[pl.ds(r0, ZBLK)],
                            o_h.at[c].at[pl.ds(r0, ZBLK)])

    fn = pl.kernel(body, out_type=out_type, mesh=mesh, scratch_types=scratch)
    return fn(dst4)[:, :N_NODES]


_BM = 1000


def _store_split(o_ref, h):
    o_ref[0] = h[:, :128]
    o_ref[1] = h[:, 128:]


def _tc_in_body(x_ref, w_ref, b_ref, o_ref):
    h = jnp.dot(x_ref[...], w_ref[...], preferred_element_type=jnp.float32)
    h = _leaky(h + b_ref[...])
    _store_split(o_ref, h)


def _tc_in(x, w, b):
    grid = (N_NODES // _BM,)
    return pl.pallas_call(
        _tc_in_body,
        grid=grid,
        in_specs=[
            pl.BlockSpec((_BM, D_FEAT), lambda i: (i, 0)),
            pl.BlockSpec((D_FEAT, D_HID), lambda i: (0, 0)),
            pl.BlockSpec((1, D_HID), lambda i: (0, 0)),
        ],
        out_specs=pl.BlockSpec((2, _BM, 128), lambda i: (0, i, 0)),
        out_shape=jax.ShapeDtypeStruct((2, N_NODES, 128), jnp.float32),
    )(x, w, b.reshape(1, -1))


def _tc_sage_body(summ_ref, deg_ref, xd_ref, w_ref, b_ref, o_ref):
    scale = 1.0 / jnp.maximum(deg_ref[:, 0:1], 1.0)
    a = jnp.concatenate(
        [summ_ref[0] * scale, summ_ref[1] * scale, xd_ref[0], xd_ref[1]],
        axis=1)
    h = jnp.dot(a, w_ref[...], preferred_element_type=jnp.float32)
    h = _leaky(h + b_ref[...])
    _store_split(o_ref, h)


def _tc_sage(summ2, deg, xd2, wl, bl, wr):
    wcat = jnp.concatenate([wl, wr], axis=0)
    grid = (N_NODES // _BM,)
    return pl.pallas_call(
        _tc_sage_body,
        grid=grid,
        in_specs=[
            pl.BlockSpec((2, _BM, 128), lambda i: (0, i, 0)),
            pl.BlockSpec((_BM, 128), lambda i: (i, 0)),
            pl.BlockSpec((2, _BM, 128), lambda i: (0, i, 0)),
            pl.BlockSpec((2 * D_HID, D_HID), lambda i: (0, 0)),
            pl.BlockSpec((1, D_HID), lambda i: (0, 0)),
        ],
        out_specs=pl.BlockSpec((2, _BM, 128), lambda i: (0, i, 0)),
        out_shape=jax.ShapeDtypeStruct((2, N_NODES, 128), jnp.float32),
    )(summ2, deg, xd2, wcat, bl.reshape(1, -1))


def _tc_sage_out_body(summ_ref, deg_ref, xd_ref, w_ref, b_ref,
                      wo_ref, bo_ref, o_ref):
    scale = 1.0 / jnp.maximum(deg_ref[:, 0:1], 1.0)
    a = jnp.concatenate(
        [summ_ref[0] * scale, summ_ref[1] * scale, xd_ref[0], xd_ref[1]],
        axis=1)
    h = jnp.dot(a, w_ref[...], preferred_element_type=jnp.float32)
    h = _leaky(h + b_ref[...])
    o_ref[...] = jnp.dot(h, wo_ref[...],
                         preferred_element_type=jnp.float32) + bo_ref[...]


def _tc_sage_out(summ2, deg, xd2, wl, bl, wr, wo, bo):
    wcat = jnp.concatenate([wl, wr], axis=0)
    grid = (N_NODES // _BM,)
    return pl.pallas_call(
        _tc_sage_out_body,
        grid=grid,
        in_specs=[
            pl.BlockSpec((2, _BM, 128), lambda i: (0, i, 0)),
            pl.BlockSpec((_BM, 128), lambda i: (i, 0)),
            pl.BlockSpec((2, _BM, 128), lambda i: (0, i, 0)),
            pl.BlockSpec((2 * D_HID, D_HID), lambda i: (0, 0)),
            pl.BlockSpec((1, D_HID), lambda i: (0, 0)),
            pl.BlockSpec((D_HID, D_OUT), lambda i: (0, 0)),
            pl.BlockSpec((1, D_OUT), lambda i: (0, 0)),
        ],
        out_specs=pl.BlockSpec((_BM, D_OUT), lambda i: (i, 0)),
        out_shape=jax.ShapeDtypeStruct((N_NODES, D_OUT), jnp.float32),
    )(summ2, deg, xd2, wcat, bl.reshape(1, -1), wo, bo.reshape(1, -1))


def kernel(x_user, x_movie, edge_index_um, edge_index_mu,
           Win_u, bin_u, Win_m, bin_m,
           Wl_um_0, bl_um_0, Wr_um_0, Wl_mu_0, bl_mu_0, Wr_mu_0,
           Wl_um_1, bl_um_1, Wr_um_1, Wl_mu_1, bl_mu_1, Wr_mu_1,
           Wout_u, bout_u, Wout_m, bout_m):
    src_um = edge_index_um[0].reshape(NS, NSTAGE, CH_STG, CHUNK)
    dst_um = edge_index_um[1].reshape(NS, NSTAGE, CH_STG, CHUNK)
    src_mu = edge_index_mu[0].reshape(NS, NSTAGE, CH_STG, CHUNK)
    dst_mu = edge_index_mu[1].reshape(NS, NSTAGE, CH_STG, CHUNK)

    zeros_blk = jnp.zeros((ZBLK, 128), jnp.float32)

    xu2 = _tc_in(x_user, Win_u, bin_u)
    xm2 = _tc_in(x_movie, Win_m, bin_m)

    deg2 = _sc_deg(jnp.stack([dst_um, dst_mu], axis=0))
    deg_m = deg2[0]
    deg_u = deg2[1]

    summ_m = _sc_agg(xu2, src_um, dst_um, zeros_blk)
    summ_u = _sc_agg(xm2, src_mu, dst_mu, zeros_blk)
    xm2 = _tc_sage(summ_m, deg_m, xm2, Wl_um_0, bl_um_0, Wr_um_0)
    xu2 = _tc_sage(summ_u, deg_u, xu2, Wl_mu_0, bl_mu_0, Wr_mu_0)

    summ_m = _sc_agg(xu2, src_um, dst_um, zeros_blk)
    summ_u = _sc_agg(xm2, src_mu, dst_mu, zeros_blk)
    out_m = _tc_sage_out(summ_m, deg_m, xm2, Wl_um_1, bl_um_1, Wr_um_1,
                         Wout_m, bout_m)
    out_u = _tc_sage_out(summ_u, deg_u, xu2, Wl_mu_1, bl_mu_1, Wr_mu_1,
                         Wout_u, bout_u)
    return (out_u, out_m)


# speedup vs baseline: 5.4965x; 1.0376x over previous
import functools

import jax
import jax.numpy as jnp
from jax import lax
from jax.experimental import pallas as pl
from jax.experimental.pallas import tpu as pltpu
from jax.experimental.pallas import tpu_sc as plsc

N_NODES = 10000
N_EDGES = 160000
D_FEAT = 256
D_HID = 256
D_OUT = 128

NC = 2
NS = 16
EDGES_PER_SUB = N_EDGES // NS
CHUNK = 50
NCHUNKS = EDGES_PER_SUB // CHUNK
N_PAD = 10240
ROWS_PER_SUB = N_PAD // NS
ZBLK = 128
NZCOPY = ROWS_PER_SUB // ZBLK
NSTAGE = 5
CH_STG = NCHUNKS // NSTAGE
NBUF = 4


def _leaky(x):
    return jnp.where(x >= 0, x, 0.01 * x)


def _sc_agg(x2, src3, dst3, zeros_h):
    mesh = plsc.VectorSubcoreMesh(core_axis_name="c", subcore_axis_name="s")
    out_type = jax.ShapeDtypeStruct((NC, N_PAD, 128), jnp.float32)
    scratch = (
        [pltpu.VMEM((2, CH_STG, CHUNK), jnp.int32),
         pltpu.VMEM((2, CH_STG, CHUNK), jnp.int32),
         pltpu.SemaphoreType.DMA,
         pltpu.SemaphoreType.DMA]
        + [pltpu.VMEM((CHUNK, 128), jnp.float32) for _ in range(NBUF)]
        + [pltpu.SemaphoreType.DMA for _ in range(NBUF)]
        + [pltpu.VMEM_SHARED((N_PAD, 128), jnp.float32)]
    )

    def body(x2_h, src_h, dst_h, z_h, o_h, src_v, dst_v, si0, si1, *rest):
        rows = rest[:NBUF]
        sems = rest[NBUF:2 * NBUF]
        acc = rest[2 * NBUF]
        si = (si0, si1)
        c = lax.axis_index("c")
        s = lax.axis_index("s")

        def _idx_start(h, p):
            pltpu.async_copy(src_h.at[s].at[h], src_v.at[p], si[p])
            pltpu.async_copy(dst_h.at[s].at[h], dst_v.at[p], si[p])

        def _idx_wait(h, p):
            pltpu.make_async_copy(src_h.at[s].at[h], src_v.at[p],
                                  si[p]).wait()
            pltpu.make_async_copy(dst_h.at[s].at[h], dst_v.at[p],
                                  si[p]).wait()

        def _gather(p, j, b):
            pltpu.async_copy(x2_h.at[c].at[src_v.at[p].at[j]],
                             rows[b], sems[b])

        def _scat(p, j, b):
            pltpu.make_async_copy(x2_h.at[c].at[src_v.at[p].at[j]],
                                  rows[b], sems[b]).wait()
            pltpu.sync_copy(rows[b], acc.at[dst_v.at[p].at[j]], add=True)

        _idx_start(0, 0)

        @pl.loop(0, NZCOPY)
        def _zacc(t):
            pltpu.sync_copy(
                z_h, acc.at[pl.ds(s * ROWS_PER_SUB + t * ZBLK, ZBLK)])

        plsc.subcore_barrier()

        for h in range(NSTAGE):
            p = h % 2
            _idx_wait(h, p)
            if h + 1 < NSTAGE:
                _idx_start(h + 1, (h + 1) % 2)

            for b in range(NBUF - 1):
                _gather(p, b, b)

            @pl.loop(0, CH_STG // NBUF)
            def _pipe(q):
                for b in range(NBUF):
                    j = NBUF * q + b
                    nxt = j + NBUF - 1

                    @pl.when(nxt < CH_STG)
                    def _():
                        _gather(p, nxt, (b + NBUF - 1) % NBUF)

                    _scat(p, j, b)

        plsc.subcore_barrier()

        @pl.loop(0, NZCOPY)
        def _wr(t):
            r0 = s * ROWS_PER_SUB + t * ZBLK
            pltpu.sync_copy(acc.at[pl.ds(r0, ZBLK)],
                            o_h.at[c].at[pl.ds(r0, ZBLK)])

    fn = pl.kernel(body, out_type=out_type, mesh=mesh, scratch_types=scratch)
    return fn(x2, src3, dst3, zeros_h)[:, :N_NODES]


def _sc_deg(dst4):
    mesh = plsc.VectorSubcoreMesh(core_axis_name="c", subcore_axis_name="s")
    out_type = jax.ShapeDtypeStruct((NC, N_PAD, 128), jnp.float32)
    scratch = [
        pltpu.VMEM((NSTAGE, CH_STG, CHUNK), jnp.int32),
        pltpu.VMEM((ZBLK, 128), jnp.float32),
        pltpu.VMEM_SHARED((N_PAD, 128), jnp.float32),
    ]

    def body(dst_h, o_h, dst_v, ones_v, acc):
        c = lax.axis_index("c")
        s = lax.axis_index("s")

        pltpu.sync_copy(dst_h.at[c].at[s], dst_v)

        @pl.loop(0, ZBLK)
        def _zrow(i):
            @pl.loop(0, 128 // 16)
            def _zcol(k):
                ones_v[i, pl.ds(k * 16, 16)] = jnp.zeros((16,), jnp.float32)

        @pl.loop(0, NZCOPY)
        def _zacc(t):
            pltpu.sync_copy(
                ones_v, acc.at[pl.ds(s * ROWS_PER_SUB + t * ZBLK, ZBLK)])

        @pl.loop(0, CHUNK)
        def _orow(i):
            @pl.loop(0, 128 // 16)
            def _ocol(k):
                ones_v[i, pl.ds(k * 16, 16)] = jnp.ones((16,), jnp.float32)

        plsc.subcore_barrier()

        @pl.loop(0, NSTAGE)
        def _stage(h):
            @pl.loop(0, CH_STG)
            def _edge(j):
                pltpu.sync_copy(ones_v.at[pl.ds(0, CHUNK)],
                                acc.at[dst_v.at[h].at[j]], add=True)

        plsc.subcore_barrier()

        @pl.loop(0, NZCOPY)
        def _wr(t):
            r0 = s * ROWS_PER_SUB + t * ZBLK
            pltpu.sync_copy(acc.at[pl.ds(r0, ZBLK)],
                            o_h.at[c].at[pl.ds(r0, ZBLK)])

    fn = pl.kernel(body, out_type=out_type, mesh=mesh, scratch_types=scratch)
    return fn(dst4)[:, :N_NODES]


_BM = 1000


def _store_split(o_ref, h):
    o_ref[0] = h[:, :128]
    o_ref[1] = h[:, 128:]


def _tc_in_body(x_ref, w_ref, b_ref, o_ref):
    h = jnp.dot(x_ref[...], w_ref[...], preferred_element_type=jnp.float32)
    h = _leaky(h + b_ref[...])
    _store_split(o_ref, h)


def _tc_in(x, w, b):
    grid = (N_NODES // _BM,)
    return pl.pallas_call(
        _tc_in_body,
        grid=grid,
        in_specs=[
            pl.BlockSpec((_BM, D_FEAT), lambda i: (i, 0)),
            pl.BlockSpec((D_FEAT, D_HID), lambda i: (0, 0)),
            pl.BlockSpec((1, D_HID), lambda i: (0, 0)),
        ],
        out_specs=pl.BlockSpec((2, _BM, 128), lambda i: (0, i, 0)),
        out_shape=jax.ShapeDtypeStruct((2, N_NODES, 128), jnp.float32),
    )(x, w, b.reshape(1, -1))


def _tc_sage_body(summ_ref, deg_ref, xd_ref, w_ref, b_ref, o_ref):
    scale = 1.0 / jnp.maximum(deg_ref[:, 0:1], 1.0)
    a = jnp.concatenate(
        [summ_ref[0] * scale, summ_ref[1] * scale, xd_ref[0], xd_ref[1]],
        axis=1)
    h = jnp.dot(a, w_ref[...], preferred_element_type=jnp.float32)
    h = _leaky(h + b_ref[...])
    _store_split(o_ref, h)


def _tc_sage(summ2, deg, xd2, wl, bl, wr):
    wcat = jnp.concatenate([wl, wr], axis=0)
    grid = (N_NODES // _BM,)
    return pl.pallas_call(
        _tc_sage_body,
        grid=grid,
        in_specs=[
            pl.BlockSpec((2, _BM, 128), lambda i: (0, i, 0)),
            pl.BlockSpec((_BM, 128), lambda i: (i, 0)),
            pl.BlockSpec((2, _BM, 128), lambda i: (0, i, 0)),
            pl.BlockSpec((2 * D_HID, D_HID), lambda i: (0, 0)),
            pl.BlockSpec((1, D_HID), lambda i: (0, 0)),
        ],
        out_specs=pl.BlockSpec((2, _BM, 128), lambda i: (0, i, 0)),
        out_shape=jax.ShapeDtypeStruct((2, N_NODES, 128), jnp.float32),
    )(summ2, deg, xd2, wcat, bl.reshape(1, -1))


def _tc_sage_out_body(summ_ref, deg_ref, xd_ref, w_ref, b_ref,
                      wo_ref, bo_ref, o_ref):
    scale = 1.0 / jnp.maximum(deg_ref[:, 0:1], 1.0)
    a = jnp.concatenate(
        [summ_ref[0] * scale, summ_ref[1] * scale, xd_ref[0], xd_ref[1]],
        axis=1)
    h = jnp.dot(a, w_ref[...], preferred_element_type=jnp.float32)
    h = _leaky(h + b_ref[...])
    o_ref[...] = jnp.dot(h, wo_ref[...],
                         preferred_element_type=jnp.float32) + bo_ref[...]


def _tc_sage_out(summ2, deg, xd2, wl, bl, wr, wo, bo):
    wcat = jnp.concatenate([wl, wr], axis=0)
    grid = (N_NODES // _BM,)
    return pl.pallas_call(
        _tc_sage_out_body,
        grid=grid,
        in_specs=[
            pl.BlockSpec((2, _BM, 128), lambda i: (0, i, 0)),
            pl.BlockSpec((_BM, 128), lambda i: (i, 0)),
            pl.BlockSpec((2, _BM, 128), lambda i: (0, i, 0)),
            pl.BlockSpec((2 * D_HID, D_HID), lambda i: (0, 0)),
            pl.BlockSpec((1, D_HID), lambda i: (0, 0)),
            pl.BlockSpec((D_HID, D_OUT), lambda i: (0, 0)),
            pl.BlockSpec((1, D_OUT), lambda i: (0, 0)),
        ],
        out_specs=pl.BlockSpec((_BM, D_OUT), lambda i: (i, 0)),
        out_shape=jax.ShapeDtypeStruct((N_NODES, D_OUT), jnp.float32),
    )(summ2, deg, xd2, wcat, bl.reshape(1, -1), wo, bo.reshape(1, -1))


def kernel(x_user, x_movie, edge_index_um, edge_index_mu,
           Win_u, bin_u, Win_m, bin_m,
           Wl_um_0, bl_um_0, Wr_um_0, Wl_mu_0, bl_mu_0, Wr_mu_0,
           Wl_um_1, bl_um_1, Wr_um_1, Wl_mu_1, bl_mu_1, Wr_mu_1,
           Wout_u, bout_u, Wout_m, bout_m):
    src_um = edge_index_um[0].reshape(NS, NSTAGE, CH_STG, CHUNK)
    dst_um = edge_index_um[1].reshape(NS, NSTAGE, CH_STG, CHUNK)
    src_mu = edge_index_mu[0].reshape(NS, NSTAGE, CH_STG, CHUNK)
    dst_mu = edge_index_mu[1].reshape(NS, NSTAGE, CH_STG, CHUNK)

    zeros_blk = jnp.zeros((ZBLK, 128), jnp.float32)

    xu2 = _tc_in(x_user, Win_u, bin_u)
    xm2 = _tc_in(x_movie, Win_m, bin_m)

    deg2 = _sc_deg(jnp.stack([dst_um, dst_mu], axis=0))
    deg_m = deg2[0]
    deg_u = deg2[1]

    summ_m = _sc_agg(xu2, src_um, dst_um, zeros_blk)
    summ_u = _sc_agg(xm2, src_mu, dst_mu, zeros_blk)
    xm2 = _tc_sage(summ_m, deg_m, xm2, Wl_um_0, bl_um_0, Wr_um_0)
    xu2 = _tc_sage(summ_u, deg_u, xu2, Wl_mu_0, bl_mu_0, Wr_mu_0)

    summ_m = _sc_agg(xu2, src_um, dst_um, zeros_blk)
    summ_u = _sc_agg(xm2, src_mu, dst_mu, zeros_blk)
    out_m = _tc_sage_out(summ_m, deg_m, xm2, Wl_um_1, bl_um_1, Wr_um_1,
                         Wout_m, bout_m)
    out_u = _tc_sage_out(summ_u, deg_u, xu2, Wl_mu_1, bl_mu_1, Wr_mu_1,
                         Wout_u, bout_u)
    return (out_u, out_m)


# speedup vs baseline: 5.5305x; 1.0062x over previous
import functools

import jax
import jax.numpy as jnp
from jax import lax
from jax.experimental import pallas as pl
from jax.experimental.pallas import tpu as pltpu
from jax.experimental.pallas import tpu_sc as plsc

N_NODES = 10000
N_EDGES = 160000
D_FEAT = 256
D_HID = 256
D_OUT = 128

NC = 2
NS = 16
EDGES_PER_SUB = N_EDGES // NS
CHUNK = 50
NCHUNKS = EDGES_PER_SUB // CHUNK
N_PAD = 10240
ROWS_PER_SUB = N_PAD // NS
ZBLK = 128
NZCOPY = ROWS_PER_SUB // ZBLK
NSTAGE = 5
CH_STG = NCHUNKS // NSTAGE
NBUF = 4


def _leaky(x):
    return jnp.where(x >= 0, x, 0.01 * x)


def _sc_agg(x2, src3, dst3, zeros_h):
    mesh = plsc.VectorSubcoreMesh(core_axis_name="c", subcore_axis_name="s")
    out_type = jax.ShapeDtypeStruct((NC, N_PAD, 128), jnp.float32)
    scratch = (
        [pltpu.VMEM((2, CH_STG, CHUNK), jnp.int32),
         pltpu.VMEM((2, CH_STG, CHUNK), jnp.int32),
         pltpu.SemaphoreType.DMA,
         pltpu.SemaphoreType.DMA]
        + [pltpu.VMEM((CHUNK, 128), jnp.float32) for _ in range(NBUF)]
        + [pltpu.SemaphoreType.DMA for _ in range(NBUF)]
        + [pltpu.VMEM_SHARED((N_PAD, 128), jnp.float32)]
    )

    def body(x2_h, src_h, dst_h, z_h, o_h, src_v, dst_v, si0, si1, *rest):
        rows = rest[:NBUF]
        sems = rest[NBUF:2 * NBUF]
        acc = rest[2 * NBUF]
        si = (si0, si1)
        c = lax.axis_index("c")
        s = lax.axis_index("s")

        def _idx_start(h, p):
            pltpu.async_copy(src_h.at[s].at[h], src_v.at[p], si[p])
            pltpu.async_copy(dst_h.at[s].at[h], dst_v.at[p], si[p])

        def _idx_wait(h, p):
            pltpu.make_async_copy(src_h.at[s].at[h], src_v.at[p],
                                  si[p]).wait()
            pltpu.make_async_copy(dst_h.at[s].at[h], dst_v.at[p],
                                  si[p]).wait()

        def _gather(p, j, b):
            pltpu.async_copy(x2_h.at[c].at[src_v.at[p].at[j]],
                             rows[b], sems[b])

        def _scat(p, j, b):
            pltpu.make_async_copy(x2_h.at[c].at[src_v.at[p].at[j]],
                                  rows[b], sems[b]).wait()
            pltpu.sync_copy(rows[b], acc.at[dst_v.at[p].at[j]], add=True)

        _idx_start(0, 0)

        @pl.loop(0, NZCOPY)
        def _zacc(t):
            pltpu.sync_copy(
                z_h, acc.at[pl.ds(s * ROWS_PER_SUB + t * ZBLK, ZBLK)])

        plsc.subcore_barrier()

        for h in range(NSTAGE):
            p = h % 2
            _idx_wait(h, p)
            if h + 1 < NSTAGE:
                _idx_start(h + 1, (h + 1) % 2)

            for b in range(NBUF - 1):
                _gather(p, b, b)

            @pl.loop(0, CH_STG // NBUF)
            def _pipe(q):
                for b in range(NBUF):
                    j = NBUF * q + b
                    nxt = j + NBUF - 1

                    @pl.when(nxt < CH_STG)
                    def _():
                        _gather(p, nxt, (b + NBUF - 1) % NBUF)

                    _scat(p, j, b)

        plsc.subcore_barrier()

        @pl.loop(0, NZCOPY)
        def _wr(t):
            r0 = s * ROWS_PER_SUB + t * ZBLK
            pltpu.sync_copy(acc.at[pl.ds(r0, ZBLK)],
                            o_h.at[c].at[pl.ds(r0, ZBLK)])

    fn = pl.kernel(body, out_type=out_type, mesh=mesh, scratch_types=scratch)
    return fn(x2, src3, dst3, zeros_h)[:, :N_NODES]


def _sc_deg(dst4):
    mesh = plsc.VectorSubcoreMesh(core_axis_name="c", subcore_axis_name="s")
    out_type = jax.ShapeDtypeStruct((NC, N_PAD, 128), jnp.float32)
    scratch = [
        pltpu.VMEM((NSTAGE, CH_STG, CHUNK), jnp.int32),
        pltpu.VMEM((ZBLK, 128), jnp.float32),
        pltpu.VMEM_SHARED((N_PAD, 128), jnp.float32),
    ]

    def body(dst_h, o_h, dst_v, ones_v, acc):
        c = lax.axis_index("c")
        s = lax.axis_index("s")

        pltpu.sync_copy(dst_h.at[c].at[s], dst_v)

        @pl.loop(0, ZBLK)
        def _zrow(i):
            @pl.loop(0, 128 // 16)
            def _zcol(k):
                ones_v[i, pl.ds(k * 16, 16)] = jnp.zeros((16,), jnp.float32)

        @pl.loop(0, NZCOPY)
        def _zacc(t):
            pltpu.sync_copy(
                ones_v, acc.at[pl.ds(s * ROWS_PER_SUB + t * ZBLK, ZBLK)])

        @pl.loop(0, CHUNK)
        def _orow(i):
            @pl.loop(0, 128 // 16)
            def _ocol(k):
                ones_v[i, pl.ds(k * 16, 16)] = jnp.ones((16,), jnp.float32)

        plsc.subcore_barrier()

        @pl.loop(0, NSTAGE)
        def _stage(h):
            @pl.loop(0, CH_STG)
            def _edge(j):
                pltpu.sync_copy(ones_v.at[pl.ds(0, CHUNK)],
                                acc.at[dst_v.at[h].at[j]], add=True)

        plsc.subcore_barrier()

        @pl.loop(0, NZCOPY)
        def _wr(t):
            r0 = s * ROWS_PER_SUB + t * ZBLK
            pltpu.sync_copy(acc.at[pl.ds(r0, ZBLK)],
                            o_h.at[c].at[pl.ds(r0, ZBLK)])

    fn = pl.kernel(body, out_type=out_type, mesh=mesh, scratch_types=scratch)
    return fn(dst4)[:, :N_NODES]


_BM = 2000


def _store_split(o_ref, h):
    o_ref[0] = h[:, :128]
    o_ref[1] = h[:, 128:]


def _tc_in_body(x_ref, w_ref, b_ref, o_ref):
    h = jnp.dot(x_ref[...], w_ref[...], preferred_element_type=jnp.float32)
    h = _leaky(h + b_ref[...])
    _store_split(o_ref, h)


def _tc_in(x, w, b):
    grid = (N_NODES // _BM,)
    return pl.pallas_call(
        _tc_in_body,
        grid=grid,
        in_specs=[
            pl.BlockSpec((_BM, D_FEAT), lambda i: (i, 0)),
            pl.BlockSpec((D_FEAT, D_HID), lambda i: (0, 0)),
            pl.BlockSpec((1, D_HID), lambda i: (0, 0)),
        ],
        out_specs=pl.BlockSpec((2, _BM, 128), lambda i: (0, i, 0)),
        out_shape=jax.ShapeDtypeStruct((2, N_NODES, 128), jnp.float32),
    )(x, w, b.reshape(1, -1))


def _tc_sage_body(summ_ref, deg_ref, xd_ref, w_ref, b_ref, o_ref):
    scale = 1.0 / jnp.maximum(deg_ref[:, 0:1], 1.0)
    a = jnp.concatenate(
        [summ_ref[0] * scale, summ_ref[1] * scale, xd_ref[0], xd_ref[1]],
        axis=1)
    h = jnp.dot(a, w_ref[...], preferred_element_type=jnp.float32)
    h = _leaky(h + b_ref[...])
    _store_split(o_ref, h)


def _tc_sage(summ2, deg, xd2, wl, bl, wr):
    wcat = jnp.concatenate([wl, wr], axis=0)
    grid = (N_NODES // _BM,)
    return pl.pallas_call(
        _tc_sage_body,
        grid=grid,
        in_specs=[
            pl.BlockSpec((2, _BM, 128), lambda i: (0, i, 0)),
            pl.BlockSpec((_BM, 128), lambda i: (i, 0)),
            pl.BlockSpec((2, _BM, 128), lambda i: (0, i, 0)),
            pl.BlockSpec((2 * D_HID, D_HID), lambda i: (0, 0)),
            pl.BlockSpec((1, D_HID), lambda i: (0, 0)),
        ],
        out_specs=pl.BlockSpec((2, _BM, 128), lambda i: (0, i, 0)),
        out_shape=jax.ShapeDtypeStruct((2, N_NODES, 128), jnp.float32),
    )(summ2, deg, xd2, wcat, bl.reshape(1, -1))


def _tc_sage_out_body(summ_ref, deg_ref, xd_ref, w_ref, b_ref,
                      wo_ref, bo_ref, o_ref):
    scale = 1.0 / jnp.maximum(deg_ref[:, 0:1], 1.0)
    a = jnp.concatenate(
        [summ_ref[0] * scale, summ_ref[1] * scale, xd_ref[0], xd_ref[1]],
        axis=1)
    h = jnp.dot(a, w_ref[...], preferred_element_type=jnp.float32)
    h = _leaky(h + b_ref[...])
    o_ref[...] = jnp.dot(h, wo_ref[...],
                         preferred_element_type=jnp.float32) + bo_ref[...]


def _tc_sage_out(summ2, deg, xd2, wl, bl, wr, wo, bo):
    wcat = jnp.concatenate([wl, wr], axis=0)
    grid = (N_NODES // _BM,)
    return pl.pallas_call(
        _tc_sage_out_body,
        grid=grid,
        in_specs=[
            pl.BlockSpec((2, _BM, 128), lambda i: (0, i, 0)),
            pl.BlockSpec((_BM, 128), lambda i: (i, 0)),
            pl.BlockSpec((2, _BM, 128), lambda i: (0, i, 0)),
            pl.BlockSpec((2 * D_HID, D_HID), lambda i: (0, 0)),
            pl.BlockSpec((1, D_HID), lambda i: (0, 0)),
            pl.BlockSpec((D_HID, D_OUT), lambda i: (0, 0)),
            pl.BlockSpec((1, D_OUT), lambda i: (0, 0)),
        ],
        out_specs=pl.BlockSpec((_BM, D_OUT), lambda i: (i, 0)),
        out_shape=jax.ShapeDtypeStruct((N_NODES, D_OUT), jnp.float32),
    )(summ2, deg, xd2, wcat, bl.reshape(1, -1), wo, bo.reshape(1, -1))


def kernel(x_user, x_movie, edge_index_um, edge_index_mu,
           Win_u, bin_u, Win_m, bin_m,
           Wl_um_0, bl_um_0, Wr_um_0, Wl_mu_0, bl_mu_0, Wr_mu_0,
           Wl_um_1, bl_um_1, Wr_um_1, Wl_mu_1, bl_mu_1, Wr_mu_1,
           Wout_u, bout_u, Wout_m, bout_m):
    src_um = edge_index_um[0].reshape(NS, NSTAGE, CH_STG, CHUNK)
    dst_um = edge_index_um[1].reshape(NS, NSTAGE, CH_STG, CHUNK)
    src_mu = edge_index_mu[0].reshape(NS, NSTAGE, CH_STG, CHUNK)
    dst_mu = edge_index_mu[1].reshape(NS, NSTAGE, CH_STG, CHUNK)

    zeros_blk = jnp.zeros((ZBLK, 128), jnp.float32)

    xu2 = _tc_in(x_user, Win_u, bin_u)
    xm2 = _tc_in(x_movie, Win_m, bin_m)

    deg2 = _sc_deg(jnp.stack([dst_um, dst_mu], axis=0))
    deg_m = deg2[0]
    deg_u = deg2[1]

    summ_m = _sc_agg(xu2, src_um, dst_um, zeros_blk)
    summ_u = _sc_agg(xm2, src_mu, dst_mu, zeros_blk)
    xm2 = _tc_sage(summ_m, deg_m, xm2, Wl_um_0, bl_um_0, Wr_um_0)
    xu2 = _tc_sage(summ_u, deg_u, xu2, Wl_mu_0, bl_mu_0, Wr_mu_0)

    summ_m = _sc_agg(xu2, src_um, dst_um, zeros_blk)
    summ_u = _sc_agg(xm2, src_mu, dst_mu, zeros_blk)
    out_m = _tc_sage_out(summ_m, deg_m, xm2, Wl_um_1, bl_um_1, Wr_um_1,
                         Wout_m, bout_m)
    out_u = _tc_sage_out(summ_u, deg_u, xu2, Wl_mu_1, bl_mu_1, Wr_mu_1,
                         Wout_u, bout_u)
    return (out_u, out_m)


# speedup vs baseline: 5.5310x; 1.0001x over previous
import functools

import jax
import jax.numpy as jnp
from jax import lax
from jax.experimental import pallas as pl
from jax.experimental.pallas import tpu as pltpu
from jax.experimental.pallas import tpu_sc as plsc

N_NODES = 10000
N_EDGES = 160000
D_FEAT = 256
D_HID = 256
D_OUT = 128

NC = 2
NS = 16
EDGES_PER_SUB = N_EDGES // NS
CHUNK = 50
NCHUNKS = EDGES_PER_SUB // CHUNK
N_PAD = 10240
ROWS_PER_SUB = N_PAD // NS
ZBLK = 128
NZCOPY = ROWS_PER_SUB // ZBLK
NSTAGE = 5
CH_STG = NCHUNKS // NSTAGE
NBUF = 4


def _leaky(x):
    return jnp.where(x >= 0, x, 0.01 * x)


def _sc_agg(x2, src3, dst3, zeros_h):
    mesh = plsc.VectorSubcoreMesh(core_axis_name="c", subcore_axis_name="s")
    out_type = jax.ShapeDtypeStruct((NC, N_PAD, 128), jnp.float32)
    scratch = (
        [pltpu.VMEM((2, CH_STG, CHUNK), jnp.int32),
         pltpu.VMEM((2, CH_STG, CHUNK), jnp.int32),
         pltpu.SemaphoreType.DMA,
         pltpu.SemaphoreType.DMA]
        + [pltpu.VMEM((CHUNK, 128), jnp.float32) for _ in range(NBUF)]
        + [pltpu.SemaphoreType.DMA for _ in range(NBUF)]
        + [pltpu.VMEM_SHARED((N_PAD, 128), jnp.float32)]
    )

    def body(x2_h, src_h, dst_h, z_h, o_h, src_v, dst_v, si0, si1, *rest):
        rows = rest[:NBUF]
        sems = rest[NBUF:2 * NBUF]
        acc = rest[2 * NBUF]
        si = (si0, si1)
        c = lax.axis_index("c")
        s = lax.axis_index("s")

        def _idx_start(h, p):
            pltpu.async_copy(src_h.at[s].at[h], src_v.at[p], si[p])
            pltpu.async_copy(dst_h.at[s].at[h], dst_v.at[p], si[p])

        def _idx_wait(h, p):
            pltpu.make_async_copy(src_h.at[s].at[h], src_v.at[p],
                                  si[p]).wait()
            pltpu.make_async_copy(dst_h.at[s].at[h], dst_v.at[p],
                                  si[p]).wait()

        def _gather(p, j, b):
            pltpu.async_copy(x2_h.at[c].at[src_v.at[p].at[j]],
                             rows[b], sems[b])

        def _scat(p, j, b):
            pltpu.make_async_copy(x2_h.at[c].at[src_v.at[p].at[j]],
                                  rows[b], sems[b]).wait()
            pltpu.sync_copy(rows[b], acc.at[dst_v.at[p].at[j]], add=True)

        _idx_start(0, 0)

        @pl.loop(0, NZCOPY)
        def _zacc(t):
            pltpu.sync_copy(
                z_h, acc.at[pl.ds(s * ROWS_PER_SUB + t * ZBLK, ZBLK)])

        plsc.subcore_barrier()

        for h in range(NSTAGE):
            p = h % 2
            _idx_wait(h, p)
            if h + 1 < NSTAGE:
                _idx_start(h + 1, (h + 1) % 2)

            for b in range(NBUF - 1):
                _gather(p, b, b)

            @pl.loop(0, CH_STG // NBUF)
            def _pipe(q):
                for b in range(NBUF):
                    j = NBUF * q + b
                    nxt = j + NBUF - 1

                    @pl.when(nxt < CH_STG)
                    def _():
                        _gather(p, nxt, (b + NBUF - 1) % NBUF)

                    _scat(p, j, b)

        plsc.subcore_barrier()

        @pl.loop(0, NZCOPY)
        def _wr(t):
            r0 = s * ROWS_PER_SUB + t * ZBLK
            pltpu.sync_copy(acc.at[pl.ds(r0, ZBLK)],
                            o_h.at[c].at[pl.ds(r0, ZBLK)])

    fn = pl.kernel(body, out_type=out_type, mesh=mesh, scratch_types=scratch)
    return fn(x2, src3, dst3, zeros_h)[:, :N_NODES]


def _sc_deg(dst4):
    mesh = plsc.VectorSubcoreMesh(core_axis_name="c", subcore_axis_name="s")
    out_type = jax.ShapeDtypeStruct((NC, N_PAD, 128), jnp.float32)
    scratch = [
        pltpu.VMEM((NSTAGE, CH_STG, CHUNK), jnp.int32),
        pltpu.VMEM((ZBLK, 128), jnp.float32),
        pltpu.VMEM_SHARED((N_PAD, 128), jnp.float32),
    ]

    def body(dst_h, o_h, dst_v, ones_v, acc):
        c = lax.axis_index("c")
        s = lax.axis_index("s")

        pltpu.sync_copy(dst_h.at[c].at[s], dst_v)

        @pl.loop(0, ZBLK)
        def _zrow(i):
            @pl.loop(0, 128 // 16)
            def _zcol(k):
                ones_v[i, pl.ds(k * 16, 16)] = jnp.zeros((16,), jnp.float32)

        @pl.loop(0, NZCOPY)
        def _zacc(t):
            pltpu.sync_copy(
                ones_v, acc.at[pl.ds(s * ROWS_PER_SUB + t * ZBLK, ZBLK)])

        @pl.loop(0, CHUNK)
        def _orow(i):
            @pl.loop(0, 128 // 16)
            def _ocol(k):
                ones_v[i, pl.ds(k * 16, 16)] = jnp.ones((16,), jnp.float32)

        plsc.subcore_barrier()

        @pl.loop(0, NSTAGE)
        def _stage(h):
            @pl.loop(0, CH_STG)
            def _edge(j):
                pltpu.sync_copy(ones_v.at[pl.ds(0, CHUNK)],
                                acc.at[dst_v.at[h].at[j]], add=True)

        plsc.subcore_barrier()

        @pl.loop(0, NZCOPY)
        def _wr(t):
            r0 = s * ROWS_PER_SUB + t * ZBLK
            pltpu.sync_copy(acc.at[pl.ds(r0, ZBLK)],
                            o_h.at[c].at[pl.ds(r0, ZBLK)])

    fn = pl.kernel(body, out_type=out_type, mesh=mesh, scratch_types=scratch)
    return fn(dst4)[:, :N_NODES]


_BM = 2000


def _store_split(o_ref, h):
    o_ref[0] = h[:, :128]
    o_ref[1] = h[:, 128:]


def _tc_in_body(x_ref, w_ref, b_ref, o_ref):
    h = jnp.dot(x_ref[...], w_ref[...], preferred_element_type=jnp.float32)
    h = _leaky(h + b_ref[...])
    _store_split(o_ref, h)


def _tc_in(x, w, b):
    grid = (N_NODES // _BM,)
    return pl.pallas_call(
        _tc_in_body,
        grid=grid,
        in_specs=[
            pl.BlockSpec((_BM, D_FEAT), lambda i: (i, 0)),
            pl.BlockSpec((D_FEAT, D_HID), lambda i: (0, 0)),
            pl.BlockSpec((1, D_HID), lambda i: (0, 0)),
        ],
        out_specs=pl.BlockSpec((2, _BM, 128), lambda i: (0, i, 0)),
        out_shape=jax.ShapeDtypeStruct((2, N_NODES, 128), jnp.float32),
    )(x, w, b.reshape(1, -1))


def _tc_sage_body(summ_ref, deg_ref, xd_ref, w_ref, b_ref, o_ref):
    scale = 1.0 / jnp.maximum(deg_ref[:, 0:1], 1.0)
    a = jnp.concatenate(
        [summ_ref[0] * scale, summ_ref[1] * scale, xd_ref[0], xd_ref[1]],
        axis=1)
    h = jnp.dot(a, w_ref[...], preferred_element_type=jnp.float32)
    h = _leaky(h + b_ref[...])
    _store_split(o_ref, h)


def _tc_sage(summ2, deg, xd2, wl, bl, wr):
    wcat = jnp.concatenate([wl, wr], axis=0)
    grid = (N_NODES // _BM,)
    return pl.pallas_call(
        _tc_sage_body,
        grid=grid,
        in_specs=[
            pl.BlockSpec((2, _BM, 128), lambda i: (0, i, 0)),
            pl.BlockSpec((_BM, 128), lambda i: (i, 0)),
            pl.BlockSpec((2, _BM, 128), lambda i: (0, i, 0)),
            pl.BlockSpec((2 * D_HID, D_HID), lambda i: (0, 0)),
            pl.BlockSpec((1, D_HID), lambda i: (0, 0)),
        ],
        out_specs=pl.BlockSpec((2, _BM, 128), lambda i: (0, i, 0)),
        out_shape=jax.ShapeDtypeStruct((2, N_NODES, 128), jnp.float32),
    )(summ2, deg, xd2, wcat, bl.reshape(1, -1))


def _tc_sage_out_body(summ_ref, deg_ref, xd_ref, w_ref, b_ref,
                      wo_ref, bo_ref, o_ref):
    scale = 1.0 / jnp.maximum(deg_ref[:, 0:1], 1.0)
    a = jnp.concatenate(
        [summ_ref[0] * scale, summ_ref[1] * scale, xd_ref[0], xd_ref[1]],
        axis=1)
    h = jnp.dot(a, w_ref[...], preferred_element_type=jnp.float32)
    h = _leaky(h + b_ref[...])
    o_ref[...] = jnp.dot(h, wo_ref[...],
                         preferred_element_type=jnp.float32) + bo_ref[...]


def _tc_sage_out(summ2, deg, xd2, wl, bl, wr, wo, bo):
    wcat = jnp.concatenate([wl, wr], axis=0)
    grid = (N_NODES // _BM,)
    return pl.pallas_call(
        _tc_sage_out_body,
        grid=grid,
        in_specs=[
            pl.BlockSpec((2, _BM, 128), lambda i: (0, i, 0)),
            pl.BlockSpec((_BM, 128), lambda i: (i, 0)),
            pl.BlockSpec((2, _BM, 128), lambda i: (0, i, 0)),
            pl.BlockSpec((2 * D_HID, D_HID), lambda i: (0, 0)),
            pl.BlockSpec((1, D_HID), lambda i: (0, 0)),
            pl.BlockSpec((D_HID, D_OUT), lambda i: (0, 0)),
            pl.BlockSpec((1, D_OUT), lambda i: (0, 0)),
        ],
        out_specs=pl.BlockSpec((_BM, D_OUT), lambda i: (i, 0)),
        out_shape=jax.ShapeDtypeStruct((N_NODES, D_OUT), jnp.float32),
    )(summ2, deg, xd2, wcat, bl.reshape(1, -1), wo, bo.reshape(1, -1))


def kernel(x_user, x_movie, edge_index_um, edge_index_mu,
           Win_u, bin_u, Win_m, bin_m,
           Wl_um_0, bl_um_0, Wr_um_0, Wl_mu_0, bl_mu_0, Wr_mu_0,
           Wl_um_1, bl_um_1, Wr_um_1, Wl_mu_1, bl_mu_1, Wr_mu_1,
           Wout_u, bout_u, Wout_m, bout_m):
    src_um = edge_index_um[0].reshape(NS, NSTAGE, CH_STG, CHUNK)
    dst_um = edge_index_um[1].reshape(NS, NSTAGE, CH_STG, CHUNK)
    src_mu = edge_index_mu[0].reshape(NS, NSTAGE, CH_STG, CHUNK)
    dst_mu = edge_index_mu[1].reshape(NS, NSTAGE, CH_STG, CHUNK)

    zeros_blk = jnp.zeros((ZBLK, 128), jnp.float32)

    deg2 = _sc_deg(jnp.stack([dst_um, dst_mu], axis=0))
    deg_m = deg2[0]
    deg_u = deg2[1]

    xu2 = _tc_in(x_user, Win_u, bin_u)
    xm2 = _tc_in(x_movie, Win_m, bin_m)

    summ_m = _sc_agg(xu2, src_um, dst_um, zeros_blk)
    summ_u = _sc_agg(xm2, src_mu, dst_mu, zeros_blk)
    xu2 = _tc_sage(summ_u, deg_u, xu2, Wl_mu_0, bl_mu_0, Wr_mu_0)
    xm2 = _tc_sage(summ_m, deg_m, xm2, Wl_um_0, bl_um_0, Wr_um_0)

    summ_m = _sc_agg(xu2, src_um, dst_um, zeros_blk)
    summ_u = _sc_agg(xm2, src_mu, dst_mu, zeros_blk)
    out_m = _tc_sage_out(summ_m, deg_m, xm2, Wl_um_1, bl_um_1, Wr_um_1,
                         Wout_m, bout_m)
    out_u = _tc_sage_out(summ_u, deg_u, xu2, Wl_mu_1, bl_mu_1, Wr_mu_1,
                         Wout_u, bout_u)
    return (out_u, out_m)


# speedup vs baseline: 5.5352x; 1.0007x over previous
import jax
import jax.numpy as jnp
from jax import lax
from jax.experimental import pallas as pl
from jax.experimental.pallas import tpu as pltpu
from jax.experimental.pallas import tpu_sc as plsc

N_NODES = 10000
N_EDGES = 160000
D_FEAT = 256
D_HID = 256
D_OUT = 128

NC = 2
NS = 16
EDGES_PER_SUB = N_EDGES // NS
CHUNK = 50
NCHUNKS = EDGES_PER_SUB // CHUNK
N_PAD = 10240
ROWS_PER_SUB = N_PAD // NS
ZBLK = 128
NZCOPY = ROWS_PER_SUB // ZBLK
NSTAGE = 5
CH_STG = NCHUNKS // NSTAGE
NBUF = 4


def _leaky(x):
    return jnp.where(x >= 0, x, 0.01 * x)


def _sc_agg(x2, src3, dst3, zeros_h):
    mesh = plsc.VectorSubcoreMesh(core_axis_name="c", subcore_axis_name="s")
    out_type = jax.ShapeDtypeStruct((NC, N_PAD, 128), jnp.float32)
    scratch = (
        [pltpu.VMEM((2, CH_STG, CHUNK), jnp.int32),
         pltpu.VMEM((2, CH_STG, CHUNK), jnp.int32),
         pltpu.SemaphoreType.DMA,
         pltpu.SemaphoreType.DMA]
        + [pltpu.VMEM((CHUNK, 128), jnp.float32) for _ in range(NBUF)]
        + [pltpu.SemaphoreType.DMA for _ in range(NBUF)]
        + [pltpu.VMEM_SHARED((N_PAD, 128), jnp.float32)]
    )

    def body(x2_h, src_h, dst_h, z_h, o_h, src_v, dst_v, si0, si1, *rest):
        rows = rest[:NBUF]
        sems = rest[NBUF:2 * NBUF]
        acc = rest[2 * NBUF]
        si = (si0, si1)
        c = lax.axis_index("c")
        s = lax.axis_index("s")

        def _idx_start(h, p):
            pltpu.async_copy(src_h.at[s].at[h], src_v.at[p], si[p])
            pltpu.async_copy(dst_h.at[s].at[h], dst_v.at[p], si[p])

        def _idx_wait(h, p):
            pltpu.make_async_copy(src_h.at[s].at[h], src_v.at[p],
                                  si[p]).wait()
            pltpu.make_async_copy(dst_h.at[s].at[h], dst_v.at[p],
                                  si[p]).wait()

        def _gather(p, j, b):
            pltpu.async_copy(x2_h.at[c].at[src_v.at[p].at[j]],
                             rows[b], sems[b])

        def _scat(p, j, b):
            pltpu.make_async_copy(x2_h.at[c].at[src_v.at[p].at[j]],
                                  rows[b], sems[b]).wait()
            pltpu.sync_copy(rows[b], acc.at[dst_v.at[p].at[j]], add=True)

        _idx_start(0, 0)

        @pl.loop(0, NZCOPY)
        def _zacc(t):
            pltpu.sync_copy(
                z_h, acc.at[pl.ds(s * ROWS_PER_SUB + t * ZBLK, ZBLK)])

        plsc.subcore_barrier()

        for h in range(NSTAGE):
            p = h % 2
            _idx_wait(h, p)
            if h + 1 < NSTAGE:
                _idx_start(h + 1, (h + 1) % 2)

            for b in range(NBUF - 1):
                _gather(p, b, b)

            @pl.loop(0, CH_STG // NBUF)
            def _pipe(q):
                for b in range(NBUF):
                    j = NBUF * q + b
                    nxt = j + NBUF - 1

                    @pl.when(nxt < CH_STG)
                    def _():
                        _gather(p, nxt, (b + NBUF - 1) % NBUF)

                    _scat(p, j, b)

        plsc.subcore_barrier()

        @pl.loop(0, NZCOPY)
        def _wr(t):
            r0 = s * ROWS_PER_SUB + t * ZBLK
            pltpu.sync_copy(acc.at[pl.ds(r0, ZBLK)],
                            o_h.at[c].at[pl.ds(r0, ZBLK)])

    fn = pl.kernel(body, out_type=out_type, mesh=mesh, scratch_types=scratch)
    return fn(x2, src3, dst3, zeros_h)[:, :N_NODES]


def _sc_deg(dst4):
    mesh = plsc.VectorSubcoreMesh(core_axis_name="c", subcore_axis_name="s")
    out_type = jax.ShapeDtypeStruct((NC, N_PAD, 128), jnp.float32)
    scratch = [
        pltpu.VMEM((NSTAGE, CH_STG, CHUNK), jnp.int32),
        pltpu.VMEM((ZBLK, 128), jnp.float32),
        pltpu.VMEM_SHARED((N_PAD, 128), jnp.float32),
    ]

    def body(dst_h, o_h, dst_v, ones_v, acc):
        c = lax.axis_index("c")
        s = lax.axis_index("s")

        pltpu.sync_copy(dst_h.at[c].at[s], dst_v)

        @pl.loop(0, ZBLK)
        def _zrow(i):
            @pl.loop(0, 128 // 16)
            def _zcol(k):
                ones_v[i, pl.ds(k * 16, 16)] = jnp.zeros((16,), jnp.float32)

        @pl.loop(0, NZCOPY)
        def _zacc(t):
            pltpu.sync_copy(
                ones_v, acc.at[pl.ds(s * ROWS_PER_SUB + t * ZBLK, ZBLK)])

        @pl.loop(0, CHUNK)
        def _orow(i):
            @pl.loop(0, 128 // 16)
            def _ocol(k):
                ones_v[i, pl.ds(k * 16, 16)] = jnp.ones((16,), jnp.float32)

        plsc.subcore_barrier()

        @pl.loop(0, NSTAGE)
        def _stage(h):
            @pl.loop(0, CH_STG)
            def _edge(j):
                pltpu.sync_copy(ones_v.at[pl.ds(0, CHUNK)],
                                acc.at[dst_v.at[h].at[j]], add=True)

        plsc.subcore_barrier()

        @pl.loop(0, NZCOPY)
        def _wr(t):
            r0 = s * ROWS_PER_SUB + t * ZBLK
            pltpu.sync_copy(acc.at[pl.ds(r0, ZBLK)],
                            o_h.at[c].at[pl.ds(r0, ZBLK)])

    fn = pl.kernel(body, out_type=out_type, mesh=mesh, scratch_types=scratch)
    return fn(dst4)[:, :N_NODES]


_BM = 2000


def _store_split(o_ref, h):
    o_ref[0] = h[:, :128]
    o_ref[1] = h[:, 128:]


def _tc_in_body(x_ref, w_ref, b_ref, o_ref):
    h = jnp.dot(x_ref[...], w_ref[...], preferred_element_type=jnp.float32)
    h = _leaky(h + b_ref[...])
    _store_split(o_ref, h)


def _tc_in(x, w, b):
    grid = (N_NODES // _BM,)
    return pl.pallas_call(
        _tc_in_body,
        grid=grid,
        in_specs=[
            pl.BlockSpec((_BM, D_FEAT), lambda i: (i, 0)),
            pl.BlockSpec((D_FEAT, D_HID), lambda i: (0, 0)),
            pl.BlockSpec((1, D_HID), lambda i: (0, 0)),
        ],
        out_specs=pl.BlockSpec((2, _BM, 128), lambda i: (0, i, 0)),
        out_shape=jax.ShapeDtypeStruct((2, N_NODES, 128), jnp.float32),
    )(x, w, b.reshape(1, -1))


def _tc_sage_body(summ_ref, deg_ref, xd_ref, w_ref, b_ref, o_ref):
    scale = 1.0 / jnp.maximum(deg_ref[:, 0:1], 1.0)
    a = jnp.concatenate(
        [summ_ref[0] * scale, summ_ref[1] * scale, xd_ref[0], xd_ref[1]],
        axis=1)
    h = jnp.dot(a, w_ref[...], preferred_element_type=jnp.float32)
    h = _leaky(h + b_ref[...])
    _store_split(o_ref, h)


def _tc_sage(summ2, deg, xd2, wl, bl, wr):
    wcat = jnp.concatenate([wl, wr], axis=0)
    grid = (N_NODES // _BM,)
    return pl.pallas_call(
        _tc_sage_body,
        grid=grid,
        in_specs=[
            pl.BlockSpec((2, _BM, 128), lambda i: (0, i, 0)),
            pl.BlockSpec((_BM, 128), lambda i: (i, 0)),
            pl.BlockSpec((2, _BM, 128), lambda i: (0, i, 0)),
            pl.BlockSpec((2 * D_HID, D_HID), lambda i: (0, 0)),
            pl.BlockSpec((1, D_HID), lambda i: (0, 0)),
        ],
        out_specs=pl.BlockSpec((2, _BM, 128), lambda i: (0, i, 0)),
        out_shape=jax.ShapeDtypeStruct((2, N_NODES, 128), jnp.float32),
    )(summ2, deg, xd2, wcat, bl.reshape(1, -1))


def _tc_sage_out_body(summ_ref, deg_ref, xd_ref, w_ref, b_ref,
                      wo_ref, bo_ref, o_ref):
    scale = 1.0 / jnp.maximum(deg_ref[:, 0:1], 1.0)
    a = jnp.concatenate(
        [summ_ref[0] * scale, summ_ref[1] * scale, xd_ref[0], xd_ref[1]],
        axis=1)
    h = jnp.dot(a, w_ref[...], preferred_element_type=jnp.float32)
    h = _leaky(h + b_ref[...])
    o_ref[...] = jnp.dot(h, wo_ref[...],
                         preferred_element_type=jnp.float32) + bo_ref[...]


def _tc_sage_out(summ2, deg, xd2, wl, bl, wr, wo, bo):
    wcat = jnp.concatenate([wl, wr], axis=0)
    grid = (N_NODES // _BM,)
    return pl.pallas_call(
        _tc_sage_out_body,
        grid=grid,
        in_specs=[
            pl.BlockSpec((2, _BM, 128), lambda i: (0, i, 0)),
            pl.BlockSpec((_BM, 128), lambda i: (i, 0)),
            pl.BlockSpec((2, _BM, 128), lambda i: (0, i, 0)),
            pl.BlockSpec((2 * D_HID, D_HID), lambda i: (0, 0)),
            pl.BlockSpec((1, D_HID), lambda i: (0, 0)),
            pl.BlockSpec((D_HID, D_OUT), lambda i: (0, 0)),
            pl.BlockSpec((1, D_OUT), lambda i: (0, 0)),
        ],
        out_specs=pl.BlockSpec((_BM, D_OUT), lambda i: (i, 0)),
        out_shape=jax.ShapeDtypeStruct((N_NODES, D_OUT), jnp.float32),
    )(summ2, deg, xd2, wcat, bl.reshape(1, -1), wo, bo.reshape(1, -1))


def kernel(x_user, x_movie, edge_index_um, edge_index_mu,
           Win_u, bin_u, Win_m, bin_m,
           Wl_um_0, bl_um_0, Wr_um_0, Wl_mu_0, bl_mu_0, Wr_mu_0,
           Wl_um_1, bl_um_1, Wr_um_1, Wl_mu_1, bl_mu_1, Wr_mu_1,
           Wout_u, bout_u, Wout_m, bout_m):
    src_um = edge_index_um[0].reshape(NS, NSTAGE, CH_STG, CHUNK)
    dst_um = edge_index_um[1].reshape(NS, NSTAGE, CH_STG, CHUNK)
    src_mu = edge_index_mu[0].reshape(NS, NSTAGE, CH_STG, CHUNK)
    dst_mu = edge_index_mu[1].reshape(NS, NSTAGE, CH_STG, CHUNK)

    zeros_blk = jnp.zeros((ZBLK, 128), jnp.float32)

    deg2 = _sc_deg(jnp.stack([dst_um, dst_mu], axis=0))
    deg_m = deg2[0]
    deg_u = deg2[1]

    xu2 = _tc_in(x_user, Win_u, bin_u)
    xm2 = _tc_in(x_movie, Win_m, bin_m)

    summ_m = _sc_agg(xu2, src_um, dst_um, zeros_blk)
    summ_u = _sc_agg(xm2, src_mu, dst_mu, zeros_blk)
    xu2 = _tc_sage(summ_u, deg_u, xu2, Wl_mu_0, bl_mu_0, Wr_mu_0)
    xm2 = _tc_sage(summ_m, deg_m, xm2, Wl_um_0, bl_um_0, Wr_um_0)

    summ_m = _sc_agg(xu2, src_um, dst_um, zeros_blk)
    summ_u = _sc_agg(xm2, src_mu, dst_mu, zeros_blk)
    out_m = _tc_sage_out(summ_m, deg_m, xm2, Wl_um_1, bl_um_1, Wr_um_1,
                         Wout_m, bout_m)
    out_u = _tc_sage_out(summ_u, deg_u, xu2, Wl_mu_1, bl_mu_1, Wr_mu_1,
                         Wout_u, bout_u)
    return (out_u, out_m)
